# baseline (speedup 1.0000x reference)
import math

import jax
import jax.numpy as jnp
from jax.experimental import pallas as pl
from jax.experimental.pallas import tpu as pltpu
from jax.experimental.pallas import tpu_sc as plsc

B = 4
L = 2048
H = 16
C = 64
D = H * C
S = 256
NS = L // S
TOPK = int(2 * math.log(L))
KPAD = 16

TILE = 256
NT = L // TILE
DC = 512
NDC = D // DC


def _corr_kernel(k_ref, q_ref, v_ref, r_ref, v2_ref, acc_scr, qh_scr, ql_scr):
    b = pl.program_id(0)
    s = pl.program_id(1)

    v2_ref[0, 0] = v_ref[0]
    v2_ref[0, 1] = v_ref[0]

    @pl.when(s == 0)
    def _split_q():
        qf = q_ref[0]
        qh = qf.astype(jnp.bfloat16)
        qh_scr[...] = qh
        ql_scr[...] = (qf - qh.astype(jnp.float32)).astype(jnp.bfloat16)

    kf32 = k_ref[0]
    kh = kf32.astype(jnp.bfloat16)
    kl = (kf32 - kh.astype(jnp.float32)).astype(jnp.bfloat16)

    ia = jax.lax.broadcasted_iota(jnp.int32, (S, S), 0)
    ib = jax.lax.broadcasted_iota(jnp.int32, (S, S), 1)
    jmat = jnp.where(ib == (S - 1) - ia, 1.0, 0.0).astype(jnp.bfloat16)
    khl = jnp.concatenate([kh, kl], axis=1)
    kfhl = jax.lax.dot_general(
        jmat, khl, (((1,), (0,)), ((), ())),
        preferred_element_type=jnp.float32)

    kfh = kfhl[:, :D].astype(jnp.bfloat16)
    kfl = kfhl[:, D:].astype(jnp.bfloat16)

    qh = qh_scr[...]
    ql = ql_scr[...]
    dims = (((1,), (1,)), ((), ()))
    mhl = jax.lax.dot_general(
        jnp.concatenate([kfh, kfl], axis=0), qh, dims,
        preferred_element_type=jnp.float32)
    m = (mhl[:S] + mhl[S:]
         + jax.lax.dot_general(kfh, ql, dims,
                               preferred_element_type=jnp.float32))
    sheared = pltpu.roll(m, 0, 1, stride=1, stride_axis=0)
    r = jnp.sum(sheared, axis=0, keepdims=True)
    vb = pltpu.roll(r, (2 * L - (s * S + S - 1)) % L, 1)

    @pl.when(s == 0)
    def _init_acc():
        acc_scr[...] = vb

    @pl.when(s > 0)
    def _add_acc():
        acc_scr[...] = acc_scr[...] + vb

    @pl.when(s == NS - 1)
    def _finish_batch():
        r_ref[pl.ds(b, 1), :] = acc_scr[...] * (1.0 / D)


NEG = jnp.float32(-3.0e38)


def _topk_sc(r):
    def body(r_hbm, idx_hbm, w_hbm, r_scr, u_scr, idx_scr, w_scr, sem):
        c = jax.lax.axis_index("c")
        s = jax.lax.axis_index("s")

        @pl.when((c == 0) & (s == 0))
        def _work():
            pltpu.async_copy(r_hbm, r_scr, sem).wait()

            @pl.loop(0, L // 16)
            def _usum(ci):
                base = ci * 16
                acc = (r_scr[0, pl.ds(base, 16)] + r_scr[1, pl.ds(base, 16)]
                       + r_scr[2, pl.ds(base, 16)] + r_scr[3, pl.ds(base, 16)])
                u_scr[pl.ds(base, 16)] = acc

            lane = jax.lax.iota(jnp.int32, 16)

            def _tree(v, op):
                for sh in (8, 4, 2, 1):
                    perm = jax.lax.rem(lane + sh, jnp.full(16, 16, jnp.int32))
                    v = op(v, v.at[perm].get(mode="promise_in_bounds"))
                return v

            ivec = jnp.zeros(16, jnp.int32)
            wrows = [jnp.where(lane < TOPK, 0.0, NEG).astype(jnp.float32)
                     for _ in range(B)]
            for i in range(TOPK):
                def _maxbody(ci, mv):
                    return jnp.maximum(mv, u_scr[pl.ds(ci * 16, 16)])
                mv = jax.lax.fori_loop(0, L // 16, _maxbody,
                                       jnp.full(16, NEG, jnp.float32))
                mv = _tree(mv, jnp.maximum)

                def _argbody(ci, cv):
                    chunk = u_scr[pl.ds(ci * 16, 16)]
                    cand = jnp.where(chunk == mv, ci * 16 + lane, L)
                    return jnp.minimum(cv, cand)
                iv = jax.lax.fori_loop(0, L // 16, _argbody,
                                       jnp.full(16, L, jnp.int32))
                iv = _tree(iv, jnp.minimum)
                idx = iv[0]

                ivec = ivec + jnp.where(lane == i, idx, 0)
                cbase = (idx // 16) * 16
                lid = idx - cbase
                uchunk = u_scr[pl.ds(cbase, 16)]
                u_scr[pl.ds(cbase, 16)] = jnp.where(lane == lid, NEG, uchunk)
                for b in range(B):
                    rchunk = r_scr[b, pl.ds(cbase, 16)]
                    wsel = jnp.where(lane == lid, rchunk, 0.0)
                    wv = _tree(wsel, jnp.add)[0]
                    wrows[b] = wrows[b] + jnp.where(lane == i, wv, 0.0)

            idx_scr[0, :] = ivec
            for b in range(B):
                row = wrows[b]
                e = jnp.exp(row - _tree(row, jnp.maximum))
                w_scr[b, :] = e / _tree(e, jnp.add)

            pltpu.async_copy(idx_scr, idx_hbm, sem).wait()
            pltpu.async_copy(w_scr, w_hbm, sem).wait()

    kern = pl.kernel(
        body,
        out_type=[
            jax.ShapeDtypeStruct((1, KPAD), jnp.int32),
            jax.ShapeDtypeStruct((B, KPAD), jnp.float32),
        ],
        mesh=plsc.VectorSubcoreMesh(core_axis_name="c", subcore_axis_name="s"),
        scratch_types=[
            pltpu.VMEM((B, L), jnp.float32),
            pltpu.VMEM((L,), jnp.float32),
            pltpu.VMEM((1, KPAD), jnp.int32),
            pltpu.VMEM((B, KPAD), jnp.float32),
            pltpu.SemaphoreType.DMA,
        ],
    )
    return kern(r)


def _topk_kernel(r_ref, idx_ref, w_ref):
    rfull = r_ref[...]
    u = jnp.sum(rfull, axis=0, keepdims=True)
    lane = jax.lax.broadcasted_iota(jnp.int32, (1, L), 1)
    laneb = jax.lax.broadcasted_iota(jnp.int32, (B, L), 1)
    cols = []
    idxs = []
    for _ in range(TOPK):
        mx = jnp.max(u)
        idx = jnp.min(jnp.where(u == mx, lane, L))
        idxs.append(idx)
        cols.append(jnp.sum(jnp.where(laneb == idx, rfull, 0.0),
                            axis=1, keepdims=True))
        u = jnp.where(lane == idx, -jnp.inf, u)
    wmat = jnp.concatenate(
        cols + [jnp.full((B, KPAD - TOPK), -jnp.inf, jnp.float32)], axis=1)
    wmax = jnp.max(wmat, axis=1, keepdims=True)
    we = jnp.exp(wmat - wmax)
    w_ref[...] = we / jnp.sum(we, axis=1, keepdims=True)

    klane = jax.lax.broadcasted_iota(jnp.int32, (1, KPAD), 1)
    ivec = jnp.zeros((1, KPAD), jnp.int32)
    for i in range(TOPK):
        ivec = ivec + jnp.where(klane == i, idxs[i], 0)
    idx_ref[...] = ivec


def _agg_kernel(idx_ref, w_ref, v2_ref, o_ref):
    b = pl.program_id(0)
    t = pl.program_id(1)
    base = t * TILE
    acc = jnp.zeros((TILE, 8, 128), jnp.float32)
    for i in range(TOPK):
        acc = acc + w_ref[b, i] * v2_ref[0, pl.ds(base + idx_ref[0, i], TILE)]
    o_ref[0] = acc


def kernel(query, key, value):
    q3 = query.reshape(B, L, D)
    k3 = key.reshape(B, L, D)
    v4 = value.reshape(B, L, 8, 128)

    r, v2d = pl.pallas_call(
        _corr_kernel,
        grid=(B, NS),
        in_specs=[
            pl.BlockSpec((1, S, D), lambda b, s: (b, s, 0)),
            pl.BlockSpec((1, L, D), lambda b, s: (b, 0, 0)),
            pl.BlockSpec((1, S, 8, 128), lambda b, s: (b, s, 0, 0)),
        ],
        out_specs=[
            pl.BlockSpec((B, L), lambda b, s: (0, 0)),
            pl.BlockSpec((1, 2, S, 8, 128), lambda b, s: (b, 0, s, 0, 0)),
        ],
        out_shape=[
            jax.ShapeDtypeStruct((B, L), jnp.float32),
            jax.ShapeDtypeStruct((B, 2, L, 8, 128), jnp.float32),
        ],
        scratch_shapes=[
            pltpu.VMEM((1, L), jnp.float32),
            pltpu.VMEM((L, D), jnp.bfloat16),
            pltpu.VMEM((L, D), jnp.bfloat16),
        ],
    )(k3, q3, v4)

    v2 = v2d.reshape(B, 2 * L, 8, 128)

    idx, w = _topk_sc(r)

    out = pl.pallas_call(
        _agg_kernel,
        grid=(B, NT),
        in_specs=[
            pl.BlockSpec(memory_space=pltpu.SMEM),
            pl.BlockSpec(memory_space=pltpu.SMEM),
            pl.BlockSpec((1, 2 * L, 8, 128), lambda b, t: (b, 0, 0, 0)),
        ],
        out_specs=pl.BlockSpec((1, TILE, 8, 128), lambda b, t: (b, t, 0, 0)),
        out_shape=jax.ShapeDtypeStruct((B, L, 8, 128), jnp.float32),
    )(idx, w, v2)

    return out.reshape(B, L, H, C)


# speedup vs baseline: 1.0002x; 1.0002x over previous
import math

import jax
import jax.numpy as jnp
from jax.experimental import pallas as pl
from jax.experimental.pallas import tpu as pltpu
from jax.experimental.pallas import tpu_sc as plsc

B = 4
L = 2048
H = 16
C = 64
D = H * C
S = 256
NS = L // S
TOPK = int(2 * math.log(L))
KPAD = 16

TILE = 256
NT = L // TILE


def _corr_kernel(k_ref, q_ref, v_ref, r_ref, v2_ref, acc_scr, qh_scr, ql_scr):
    b = pl.program_id(0)
    s = pl.program_id(1)

    v2_ref[0, 0] = v_ref[0]
    v2_ref[0, 1] = v_ref[0]

    @pl.when(s == 0)
    def _split_q():
        qf = q_ref[0]
        qh = qf.astype(jnp.bfloat16)
        qh_scr[...] = qh
        ql_scr[...] = (qf - qh.astype(jnp.float32)).astype(jnp.bfloat16)

    kf32 = k_ref[0]
    kh = kf32.astype(jnp.bfloat16)
    kl = (kf32 - kh.astype(jnp.float32)).astype(jnp.bfloat16)

    ia = jax.lax.broadcasted_iota(jnp.int32, (S, S), 0)
    ib = jax.lax.broadcasted_iota(jnp.int32, (S, S), 1)
    jmat = jnp.where(ib == (S - 1) - ia, 1.0, 0.0).astype(jnp.bfloat16)
    khl = jnp.concatenate([kh, kl], axis=1)
    kfhl = jax.lax.dot_general(
        jmat, khl, (((1,), (0,)), ((), ())),
        preferred_element_type=jnp.float32)

    kfh = kfhl[:, :D].astype(jnp.bfloat16)
    kfl = kfhl[:, D:].astype(jnp.bfloat16)

    qh = qh_scr[...]
    ql = ql_scr[...]
    dims = (((1,), (1,)), ((), ()))
    mhl = jax.lax.dot_general(
        jnp.concatenate([kfh, kfl], axis=0), qh, dims,
        preferred_element_type=jnp.float32)
    m = (mhl[:S] + mhl[S:]
         + jax.lax.dot_general(kfh, ql, dims,
                               preferred_element_type=jnp.float32))
    sheared = pltpu.roll(m, 0, 1, stride=1, stride_axis=0)
    r = jnp.sum(sheared, axis=0, keepdims=True)
    vb = pltpu.roll(r, (2 * L - (s * S + S - 1)) % L, 1)

    @pl.when(s == 0)
    def _init_acc():
        acc_scr[...] = vb

    @pl.when(s > 0)
    def _add_acc():
        acc_scr[...] = acc_scr[...] + vb

    @pl.when(s == NS - 1)
    def _finish_batch():
        r_ref[pl.ds(b, 1), :] = acc_scr[...] * (1.0 / D)


NEG = jnp.float32(-3.0e38)


def _topk_sc(r):
    def body(r_hbm, idx_hbm, w_hbm, r_scr, u_scr, idx_scr, w_scr, sem):
        c = jax.lax.axis_index("c")
        s = jax.lax.axis_index("s")

        @pl.when((c == 0) & (s == 0))
        def _work():
            pltpu.async_copy(r_hbm, r_scr, sem).wait()

            @pl.loop(0, L // 16)
            def _usum(ci):
                base = ci * 16
                acc = (r_scr[0, pl.ds(base, 16)] + r_scr[1, pl.ds(base, 16)]
                       + r_scr[2, pl.ds(base, 16)] + r_scr[3, pl.ds(base, 16)])
                u_scr[pl.ds(base, 16)] = acc

            lane = jax.lax.iota(jnp.int32, 16)

            def _tree(v, op):
                for sh in (8, 4, 2, 1):
                    perm = jax.lax.rem(lane + sh, jnp.full(16, 16, jnp.int32))
                    v = op(v, v.at[perm].get(mode="promise_in_bounds"))
                return v

            ivec = jnp.zeros(16, jnp.int32)
            wrows = [jnp.where(lane < TOPK, 0.0, NEG).astype(jnp.float32)
                     for _ in range(B)]
            for i in range(TOPK):
                def _maxbody(ci, mv):
                    return jnp.maximum(mv, u_scr[pl.ds(ci * 16, 16)])
                mv = jax.lax.fori_loop(0, L // 16, _maxbody,
                                       jnp.full(16, NEG, jnp.float32))
                mv = _tree(mv, jnp.maximum)

                def _argbody(ci, cv):
                    chunk = u_scr[pl.ds(ci * 16, 16)]
                    cand = jnp.where(chunk == mv, ci * 16 + lane, L)
                    return jnp.minimum(cv, cand)
                iv = jax.lax.fori_loop(0, L // 16, _argbody,
                                       jnp.full(16, L, jnp.int32))
                iv = _tree(iv, jnp.minimum)
                idx = iv[0]

                ivec = ivec + jnp.where(lane == i, idx, 0)
                cbase = (idx // 16) * 16
                lid = idx - cbase
                uchunk = u_scr[pl.ds(cbase, 16)]
                u_scr[pl.ds(cbase, 16)] = jnp.where(lane == lid, NEG, uchunk)
                for b in range(B):
                    rchunk = r_scr[b, pl.ds(cbase, 16)]
                    wsel = jnp.where(lane == lid, rchunk, 0.0)
                    wv = _tree(wsel, jnp.add)[0]
                    wrows[b] = wrows[b] + jnp.where(lane == i, wv, 0.0)

            idx_scr[0, :] = ivec
            for b in range(B):
                row = wrows[b]
                e = jnp.exp(row - _tree(row, jnp.maximum))
                w_scr[b, :] = e / _tree(e, jnp.add)

            pltpu.async_copy(idx_scr, idx_hbm, sem).wait()
            pltpu.async_copy(w_scr, w_hbm, sem).wait()

    kern = pl.kernel(
        body,
        out_type=[
            jax.ShapeDtypeStruct((1, KPAD), jnp.int32),
            jax.ShapeDtypeStruct((B, KPAD), jnp.float32),
        ],
        mesh=plsc.VectorSubcoreMesh(core_axis_name="c", subcore_axis_name="s"),
        scratch_types=[
            pltpu.VMEM((B, L), jnp.float32),
            pltpu.VMEM((L,), jnp.float32),
            pltpu.VMEM((1, KPAD), jnp.int32),
            pltpu.VMEM((B, KPAD), jnp.float32),
            pltpu.SemaphoreType.DMA,
        ],
    )
    return kern(r)


def _agg_kernel(idx_ref, w_ref, v2_ref, o_ref):
    b = pl.program_id(0)
    t = pl.program_id(1)
    base = t * TILE
    acc = jnp.zeros((TILE, 8, 128), jnp.float32)
    for i in range(TOPK):
        acc = acc + w_ref[b, i] * v2_ref[0, pl.ds(base + idx_ref[0, i], TILE)]
    o_ref[0] = acc


def kernel(query, key, value):
    q3 = query.reshape(B, L, D)
    k3 = key.reshape(B, L, D)
    v4 = value.reshape(B, L, 8, 128)

    r, v2d = pl.pallas_call(
        _corr_kernel,
        grid=(B, NS),
        in_specs=[
            pl.BlockSpec((1, S, D), lambda b, s: (b, s, 0)),
            pl.BlockSpec((1, L, D), lambda b, s: (b, 0, 0)),
            pl.BlockSpec((1, S, 8, 128), lambda b, s: (b, s, 0, 0)),
        ],
        out_specs=[
            pl.BlockSpec((B, L), lambda b, s: (0, 0)),
            pl.BlockSpec((1, 2, S, 8, 128), lambda b, s: (b, 0, s, 0, 0)),
        ],
        out_shape=[
            jax.ShapeDtypeStruct((B, L), jnp.float32),
            jax.ShapeDtypeStruct((B, 2, L, 8, 128), jnp.float32),
        ],
        scratch_shapes=[
            pltpu.VMEM((1, L), jnp.float32),
            pltpu.VMEM((L, D), jnp.bfloat16),
            pltpu.VMEM((L, D), jnp.bfloat16),
        ],
    )(k3, q3, v4)

    v2 = v2d.reshape(B, 2 * L, 8, 128)

    idx, w = _topk_sc(r)

    out = pl.pallas_call(
        _agg_kernel,
        grid=(B, NT),
        in_specs=[
            pl.BlockSpec(memory_space=pltpu.SMEM),
            pl.BlockSpec(memory_space=pltpu.SMEM),
            pl.BlockSpec((1, 2 * L, 8, 128), lambda b, t: (b, 0, 0, 0)),
        ],
        out_specs=pl.BlockSpec((1, TILE, 8, 128), lambda b, t: (b, t, 0, 0)),
        out_shape=jax.ShapeDtypeStruct((B, L, 8, 128), jnp.float32),
    )(idx, w, v2)

    return out.reshape(B, L, H, C)
